# baseline (speedup 1.0000x reference)
import functools

import jax
import jax.numpy as jnp
from jax import lax
from jax.experimental import pallas as pl
from jax.experimental.pallas import tpu as pltpu
from jax.experimental.pallas import tpu_sc as plsc

B, C, H, W = 32, 768, 32, 32
N = H * W
R = B * C
K = 205
ALPHA = 0.7

NC, NS, L = 2, 16, 16
NW = NC * NS
RPW = R // NW
GROUP = 16
NGRP = RPW // GROUP
CH32 = N // (2 * L)
NBITS = 14


def _kernel_body(x_hbm, out_hbm, xbuf, kbuf, outbuf):
    wid = lax.axis_index("s") * NC + lax.axis_index("c")
    zero = jnp.zeros((L,), jnp.int32)
    one = jnp.ones((L,), jnp.int32)
    fzero = jnp.zeros((L,), jnp.float32)
    bzero = jnp.zeros((2 * L,), jnp.bfloat16)
    bone = jnp.ones((2 * L,), jnp.bfloat16)
    lanes = lax.iota(jnp.int32, L)

    def u2bits(u):
        return jnp.where(u >= 32768, u - 32768, 65535 - u)

    def u2f32vec(u):
        return plsc.bitcast(jnp.full((L,), u2bits(u) << 16, jnp.int32),
                            jnp.float32)

    def u2bfvec(u):
        b = u2bits(u)
        return plsc.bitcast(jnp.full((L,), b | (b << 16), jnp.int32),
                            jnp.bfloat16)

    def group_body(g, carry):
        row0 = wid * RPW + g * GROUP
        pltpu.sync_copy(x_hbm.at[pl.ds(row0 * N, GROUP * N)], xbuf)

        def key_body(j, c):
            for u in range(4):
                off = j * (8 * L) + u * (2 * L)
                a = xbuf[pl.ds(off, L)]
                b = xbuf[pl.ds(off + L, L)]
                p = plsc.pack(a, b, format=plsc.PackFormat.INTERLEAVED)
                kbuf[pl.ds(off // 2, L)] = plsc.bitcast(p, jnp.int32)
            return c

        lax.fori_loop(0, GROUP * N // (8 * L), key_body, 0)

        def quad_body(q, ovec):
            bases = [(q * 4 + rr) * (N // 2) for rr in range(4)]

            def bit_body(i, st):
                t1 = st[0:4]
                t2 = st[4:8]
                bit = st[8]
                cand1 = [t1[rr] + bit for rr in range(4)]
                cand2 = [t2[rr] + bit for rr in range(4)]
                cv1 = [u2bfvec(cand1[rr]) for rr in range(4)]
                cv2 = [u2bfvec(65535 - cand2[rr]) for rr in range(4)]

                def cbody(j, c):
                    c = list(c)
                    for u in range(8):
                        for rr in range(4):
                            v = plsc.bitcast(
                                kbuf[pl.ds(bases[rr] + (j * 8 + u) * L, L)],
                                jnp.bfloat16)
                            c[rr] = c[rr] + jnp.where(v >= cv1[rr],
                                                      bone, bzero)
                            c[4 + rr] = c[4 + rr] + jnp.where(v <= cv2[rr],
                                                              bone, bzero)
                    return tuple(c)

                cs = lax.fori_loop(0, CH32 // 8, cbody, (bzero,) * 8)
                nt, nb = [], []
                for rr in range(4):
                    a1, b1 = plsc.unpack(cs[rr],
                                         format=plsc.PackFormat.INTERLEAVED)
                    a2, b2 = plsc.unpack(cs[4 + rr],
                                         format=plsc.PackFormat.INTERLEAVED)
                    nt.append(jnp.sum(a1 + b1))
                    nb.append(jnp.sum(a2 + b2))
                new1 = tuple(jnp.where(nt[rr] >= float(K), cand1[rr], t1[rr])
                             for rr in range(4))
                new2 = tuple(jnp.where(nb[rr] >= float(K), cand2[rr], t2[rr])
                             for rr in range(4))
                return new1 + new2 + (bit >> 1,)

            st = lax.fori_loop(
                0, NBITS, bit_body,
                (jnp.int32(0),) * 8 + (jnp.int32(32768),))

            for rr in range(4):
                r = q * 4 + rr
                base = r * N
                t1 = st[rr]
                bot = 65535 - st[4 + rr]
                val_top = 0.5 * (u2f32vec(t1) + u2f32vec(t1 + 3))
                val_bot = 0.5 * (u2f32vec(bot - 3) + u2f32vec(bot))
                ub = 0.5 * (u2f32vec(t1 + 3) + u2f32vec(t1 + 4))
                lb = 0.5 * (u2f32vec(bot - 4) + u2f32vec(bot - 3))

                def fbody(j, c):
                    cg, sg, cl, sl = c
                    for u in range(8):
                        xv = xbuf[pl.ds(base + (j * 8 + u) * L, L)]
                        m1 = xv > ub
                        m2 = xv < lb
                        cg = cg + jnp.where(m1, one, zero)
                        sg = sg + jnp.where(m1, xv, fzero)
                        cl = cl + jnp.where(m2, one, zero)
                        sl = sl + jnp.where(m2, xv, fzero)
                    return cg, sg, cl, sl

                cg, sg, cl, sl = lax.fori_loop(
                    0, N // (8 * L), fbody, (zero, fzero, zero, fzero))

                ng = jnp.full((L,), K - jnp.sum(cg),
                              jnp.int32).astype(jnp.float32)
                nl = jnp.full((L,), K - jnp.sum(cl),
                              jnp.int32).astype(jnp.float32)
                sgv = jnp.full((L,), jnp.sum(sg), jnp.float32)
                slv = jnp.full((L,), jnp.sum(sl), jnp.float32)
                top_sum = sgv + ng * val_top
                bot_sum = slv + nl * val_bot
                outv = (top_sum * (1.0 / (2 * K))
                        + bot_sum * (ALPHA / (2 * K)))
                ovec = jnp.where(lanes == r, outv, ovec)
            return ovec

        ovec = lax.fori_loop(0, GROUP // 4, quad_body, fzero)
        outbuf[pl.ds(g * GROUP, GROUP)] = ovec
        return carry

    lax.fori_loop(0, NGRP, group_body, 0)
    pltpu.sync_copy(outbuf, out_hbm.at[pl.ds(wid * RPW, RPW)])


@jax.jit
def kernel(input):
    x = input.reshape(R * N)
    mesh = plsc.VectorSubcoreMesh(
        core_axis_name="c", subcore_axis_name="s",
        num_cores=NC, num_subcores=NS)
    out = pl.kernel(
        _kernel_body,
        out_type=jax.ShapeDtypeStruct((R,), jnp.float32),
        mesh=mesh,
        compiler_params=pltpu.CompilerParams(needs_layout_passes=False),
        scratch_types=[
            pltpu.VMEM((GROUP * N,), jnp.float32),
            pltpu.VMEM((GROUP * N // 2,), jnp.int32),
            pltpu.VMEM((RPW,), jnp.float32),
        ],
    )(x)
    return out.reshape(B, C)


# speedup vs baseline: 1.2297x; 1.2297x over previous
import functools

import jax
import jax.numpy as jnp
from jax import lax
from jax.experimental import pallas as pl
from jax.experimental.pallas import tpu as pltpu
from jax.experimental.pallas import tpu_sc as plsc

B, C, H, W = 32, 768, 32, 32
N = H * W
R = B * C
K = 205
ALPHA = 0.7

NC, NS, L = 2, 16, 16
NW = NC * NS
RPW = R // NW
GROUP = 16
NGRP = RPW // GROUP
CH32 = N // (2 * L)
NBITS = 14


def _kernel_body(x_hbm, out_hbm, xbuf, kbuf, outbuf):
    wid = lax.axis_index("s") * NC + lax.axis_index("c")
    zero = jnp.zeros((L,), jnp.int32)
    one = jnp.ones((L,), jnp.int32)
    fzero = jnp.zeros((L,), jnp.float32)
    bzero = jnp.zeros((2 * L,), jnp.bfloat16)
    bone = jnp.ones((2 * L,), jnp.bfloat16)
    lanes = lax.iota(jnp.int32, L)

    def u2bits(u):
        return jnp.where(u >= 32768, u - 32768, 65535 - u)

    def u2f32vec(u):
        return plsc.bitcast(jnp.full((L,), u2bits(u) << 16, jnp.int32),
                            jnp.float32)

    def u2bfvec(u):
        b = u2bits(u)
        return plsc.bitcast(jnp.full((L,), b | (b << 16), jnp.int32),
                            jnp.bfloat16)

    def group_body(g, carry):
        row0 = wid * RPW + g * GROUP
        pltpu.sync_copy(x_hbm.at[pl.ds(row0 * N, GROUP * N)], xbuf)

        def key_body(j, c):
            for u in range(4):
                off = j * (8 * L) + u * (2 * L)
                a = xbuf[pl.ds(off, L)]
                b = xbuf[pl.ds(off + L, L)]
                p = plsc.pack(a, b, format=plsc.PackFormat.INTERLEAVED)
                kbuf[pl.ds(off // 2, L)] = plsc.bitcast(p, jnp.int32)
            return c

        lax.fori_loop(0, GROUP * N // (8 * L), key_body, 0)

        def quad_body(q, ovec):
            NR = 2
            bases = [(q * NR + rr) * (N // 2) for rr in range(NR)]

            def bit_body(i, st):
                t1 = st[0:NR]
                t2 = st[NR:2 * NR]
                bit = st[2 * NR]
                cand1 = [t1[rr] + bit for rr in range(NR)]
                cand2 = [t2[rr] + bit for rr in range(NR)]
                cv1 = [u2bfvec(cand1[rr]) for rr in range(NR)]
                cv2 = [u2bfvec(65535 - cand2[rr]) for rr in range(NR)]

                def cbody(j, c):
                    c = list(c)
                    for u in range(8):
                        for rr in range(NR):
                            v = plsc.bitcast(
                                kbuf[pl.ds(bases[rr] + (j * 8 + u) * L, L)],
                                jnp.bfloat16)
                            c[rr] = c[rr] + jnp.where(v >= cv1[rr],
                                                      bone, bzero)
                            c[NR + rr] = c[NR + rr] + jnp.where(
                                v <= cv2[rr], bone, bzero)
                    return tuple(c)

                cs = lax.fori_loop(0, CH32 // 8, cbody, (bzero,) * (2 * NR))
                nt, nb = [], []
                for rr in range(NR):
                    a1, b1 = plsc.unpack(cs[rr],
                                         format=plsc.PackFormat.INTERLEAVED)
                    a2, b2 = plsc.unpack(cs[NR + rr],
                                         format=plsc.PackFormat.INTERLEAVED)
                    nt.append(jnp.sum(a1 + b1))
                    nb.append(jnp.sum(a2 + b2))
                new1 = tuple(jnp.where(nt[rr] >= float(K), cand1[rr], t1[rr])
                             for rr in range(NR))
                new2 = tuple(jnp.where(nb[rr] >= float(K), cand2[rr], t2[rr])
                             for rr in range(NR))
                return new1 + new2 + (bit >> 1,)

            st = lax.fori_loop(
                0, NBITS, bit_body,
                (jnp.int32(0),) * (2 * NR) + (jnp.int32(32768),))

            for rr in range(2):
                r = q * 2 + rr
                base = r * N
                t1 = st[rr]
                bot = 65535 - st[2 + rr]
                val_top = 0.5 * (u2f32vec(t1) + u2f32vec(t1 + 3))
                val_bot = 0.5 * (u2f32vec(bot - 3) + u2f32vec(bot))
                ub = 0.5 * (u2f32vec(t1 + 3) + u2f32vec(t1 + 4))
                lb = 0.5 * (u2f32vec(bot - 4) + u2f32vec(bot - 3))

                def fbody(j, c):
                    cg, sg, cl, sl = c
                    for u in range(8):
                        xv = xbuf[pl.ds(base + (j * 8 + u) * L, L)]
                        m1 = xv > ub
                        m2 = xv < lb
                        cg = cg + jnp.where(m1, one, zero)
                        sg = sg + jnp.where(m1, xv, fzero)
                        cl = cl + jnp.where(m2, one, zero)
                        sl = sl + jnp.where(m2, xv, fzero)
                    return cg, sg, cl, sl

                cg, sg, cl, sl = lax.fori_loop(
                    0, N // (8 * L), fbody, (zero, fzero, zero, fzero))

                ng = jnp.full((L,), K - jnp.sum(cg),
                              jnp.int32).astype(jnp.float32)
                nl = jnp.full((L,), K - jnp.sum(cl),
                              jnp.int32).astype(jnp.float32)
                sgv = jnp.full((L,), jnp.sum(sg), jnp.float32)
                slv = jnp.full((L,), jnp.sum(sl), jnp.float32)
                top_sum = sgv + ng * val_top
                bot_sum = slv + nl * val_bot
                outv = (top_sum * (1.0 / (2 * K))
                        + bot_sum * (ALPHA / (2 * K)))
                ovec = jnp.where(lanes == r, outv, ovec)
            return ovec

        ovec = lax.fori_loop(0, GROUP // 2, quad_body, fzero)
        outbuf[pl.ds(g * GROUP, GROUP)] = ovec
        return carry

    lax.fori_loop(0, NGRP, group_body, 0)
    pltpu.sync_copy(outbuf, out_hbm.at[pl.ds(wid * RPW, RPW)])


@jax.jit
def kernel(input):
    x = input.reshape(R * N)
    mesh = plsc.VectorSubcoreMesh(
        core_axis_name="c", subcore_axis_name="s",
        num_cores=NC, num_subcores=NS)
    out = pl.kernel(
        _kernel_body,
        out_type=jax.ShapeDtypeStruct((R,), jnp.float32),
        mesh=mesh,
        compiler_params=pltpu.CompilerParams(needs_layout_passes=False),
        scratch_types=[
            pltpu.VMEM((GROUP * N,), jnp.float32),
            pltpu.VMEM((GROUP * N // 2,), jnp.int32),
            pltpu.VMEM((RPW,), jnp.float32),
        ],
    )(x)
    return out.reshape(B, C)


# speedup vs baseline: 1.4139x; 1.1498x over previous
import functools

import jax
import jax.numpy as jnp
from jax import lax
from jax.experimental import pallas as pl
from jax.experimental.pallas import tpu as pltpu
from jax.experimental.pallas import tpu_sc as plsc

B, C, H, W = 32, 768, 32, 32
N = H * W
R = B * C
K = 205
ALPHA = 0.7

NC, NS, L = 2, 16, 16
NW = NC * NS
RPW = R // NW
GROUP = 16
NGRP = RPW // GROUP
CH32 = N // (2 * L)
NBITS = 14

_DNUMS = lax.GatherDimensionNumbers(
    offset_dims=(), collapsed_slice_dims=(0,), start_index_map=(0,))


def _permute(v, p):
    return lax.gather(v, p[:, None], dimension_numbers=_DNUMS,
                      slice_sizes=(1,),
                      mode=lax.GatherScatterMode.PROMISE_IN_BOUNDS)


def _kernel_body(x_hbm, out_hbm, xbuf, kbuf, outbuf):
    wid = lax.axis_index("s") * NC + lax.axis_index("c")
    zero = jnp.zeros((L,), jnp.int32)
    one = jnp.ones((L,), jnp.int32)
    fzero = jnp.zeros((L,), jnp.float32)
    bzero = jnp.zeros((2 * L,), jnp.bfloat16)
    bone = jnp.ones((2 * L,), jnp.bfloat16)
    lanes = lax.iota(jnp.int32, L)
    perms = [lanes ^ sh for sh in (8, 4, 2, 1)]

    def allsum(v):
        for p in perms:
            v = v + _permute(v, p)
        return v

    def u2bits(u):
        return jnp.where(u >= 32768, u - 32768, 65535 - u)

    def u2f32(u):
        return plsc.bitcast(u2bits(u) << 16, jnp.float32)

    def u2bf(u):
        b = u2bits(u)
        return plsc.bitcast(b | (b << 16), jnp.bfloat16)

    def group_body(g, carry):
        row0 = wid * RPW + g * GROUP
        pltpu.sync_copy(x_hbm.at[pl.ds(row0 * N, GROUP * N)], xbuf)

        def key_body(j, c):
            for u in range(4):
                off = j * (8 * L) + u * (2 * L)
                a = xbuf[pl.ds(off, L)]
                b = xbuf[pl.ds(off + L, L)]
                p = plsc.pack(a, b, format=plsc.PackFormat.INTERLEAVED)
                kbuf[pl.ds(off // 2, L)] = plsc.bitcast(p, jnp.int32)
            return c

        lax.fori_loop(0, GROUP * N // (8 * L), key_body, 0)

        def row_body(r, ovec):
            kbase = r * (N // 2)
            base = r * N

            def bit_body(i, st):
                t1v, t2v, bitv = st
                cand1 = t1v + bitv
                cand2 = t2v + bitv
                cv1 = u2bf(cand1)
                cv2 = u2bf(65535 - cand2)

                def cbody(j, c):
                    c1a, c1b, c2a, c2b = c
                    for u in range(8):
                        v = plsc.bitcast(
                            kbuf[pl.ds(kbase + (j * 8 + u) * L, L)],
                            jnp.bfloat16)
                        i1 = jnp.where(v >= cv1, bone, bzero)
                        i2 = jnp.where(v <= cv2, bone, bzero)
                        if u % 2 == 0:
                            c1a = c1a + i1
                            c2a = c2a + i2
                        else:
                            c1b = c1b + i1
                            c2b = c2b + i2
                    return c1a, c1b, c2a, c2b

                c1a, c1b, c2a, c2b = lax.fori_loop(
                    0, CH32 // 8, cbody, (bzero, bzero, bzero, bzero))
                u1a, u1b = plsc.unpack(c1a + c1b,
                                       format=plsc.PackFormat.INTERLEAVED)
                u2a, u2b = plsc.unpack(c2a + c2b,
                                       format=plsc.PackFormat.INTERLEAVED)
                n1 = allsum(u1a + u1b)
                n2 = allsum(u2a + u2b)
                t1v = jnp.where(n1 >= float(K), cand1, t1v)
                t2v = jnp.where(n2 >= float(K), cand2, t2v)
                return t1v, t2v, bitv >> 1

            t1, t2, _ = lax.fori_loop(
                0, NBITS, bit_body,
                (zero, zero, jnp.full((L,), 32768, jnp.int32)))
            bot = 65535 - t2

            val_top = 0.5 * (u2f32(t1) + u2f32(t1 + 3))
            val_bot = 0.5 * (u2f32(bot - 3) + u2f32(bot))
            ub = 0.5 * (u2f32(t1 + 3) + u2f32(t1 + 4))
            lb = 0.5 * (u2f32(bot - 4) + u2f32(bot - 3))

            def fbody(j, c):
                cg, sg, cl, sl = c
                for u in range(8):
                    xv = xbuf[pl.ds(base + (j * 8 + u) * L, L)]
                    m1 = xv > ub
                    m2 = xv < lb
                    cg = cg + jnp.where(m1, one, zero)
                    sg = sg + jnp.where(m1, xv, fzero)
                    cl = cl + jnp.where(m2, one, zero)
                    sl = sl + jnp.where(m2, xv, fzero)
                return cg, sg, cl, sl

            cg, sg, cl, sl = lax.fori_loop(
                0, N // (8 * L), fbody, (zero, fzero, zero, fzero))

            ng = float(K) - allsum(cg.astype(jnp.float32))
            nl = float(K) - allsum(cl.astype(jnp.float32))
            sgv = allsum(sg)
            slv = allsum(sl)
            top_sum = sgv + ng * val_top
            bot_sum = slv + nl * val_bot
            outv = top_sum * (1.0 / (2 * K)) + bot_sum * (ALPHA / (2 * K))
            return jnp.where(lanes == r, outv, ovec)

        ovec = lax.fori_loop(0, GROUP, row_body, fzero)
        outbuf[pl.ds(g * GROUP, GROUP)] = ovec
        return carry

    lax.fori_loop(0, NGRP, group_body, 0)
    pltpu.sync_copy(outbuf, out_hbm.at[pl.ds(wid * RPW, RPW)])


@jax.jit
def kernel(input):
    x = input.reshape(R * N)
    mesh = plsc.VectorSubcoreMesh(
        core_axis_name="c", subcore_axis_name="s",
        num_cores=NC, num_subcores=NS)
    out = pl.kernel(
        _kernel_body,
        out_type=jax.ShapeDtypeStruct((R,), jnp.float32),
        mesh=mesh,
        compiler_params=pltpu.CompilerParams(needs_layout_passes=False),
        scratch_types=[
            pltpu.VMEM((GROUP * N,), jnp.float32),
            pltpu.VMEM((GROUP * N // 2,), jnp.int32),
            pltpu.VMEM((RPW,), jnp.float32),
        ],
    )(x)
    return out.reshape(B, C)


# speedup vs baseline: 1.5255x; 1.0789x over previous
import functools

import jax
import jax.numpy as jnp
from jax import lax
from jax.experimental import pallas as pl
from jax.experimental.pallas import tpu as pltpu
from jax.experimental.pallas import tpu_sc as plsc

B, C, H, W = 32, 768, 32, 32
N = H * W
R = B * C
K = 205
ALPHA = 0.7

NC, NS, L = 2, 16, 16
NW = NC * NS
RPW = R // NW
GROUP = 16
NGRP = RPW // GROUP
CH32 = N // (2 * L)
NBITS = 14

_DNUMS = lax.GatherDimensionNumbers(
    offset_dims=(), collapsed_slice_dims=(0,), start_index_map=(0,))


def _permute(v, p):
    return lax.gather(v, p[:, None], dimension_numbers=_DNUMS,
                      slice_sizes=(1,),
                      mode=lax.GatherScatterMode.PROMISE_IN_BOUNDS)


def _kernel_body(x_hbm, out_hbm, xbuf, kbuf, outbuf, sem0, sem1):
    wid = lax.axis_index("s") * NC + lax.axis_index("c")
    zero = jnp.zeros((L,), jnp.int32)
    one = jnp.ones((L,), jnp.int32)
    fzero = jnp.zeros((L,), jnp.float32)
    bzero = jnp.zeros((2 * L,), jnp.bfloat16)
    bone = jnp.ones((2 * L,), jnp.bfloat16)
    lanes = lax.iota(jnp.int32, L)
    perms = [lanes ^ sh for sh in (8, 4, 2, 1)]

    def allsum(v):
        for p in perms:
            v = v + _permute(v, p)
        return v

    def u2bits(u):
        return jnp.where(u >= 32768, u - 32768, 65535 - u)

    def u2f32(u):
        return plsc.bitcast(u2bits(u) << 16, jnp.float32)

    def u2bf(u):
        b = u2bits(u)
        return plsc.bitcast(b | (b << 16), jnp.bfloat16)

    GN = GROUP * N

    def copy_in(g, buf_i, sem):
        row0 = wid * RPW + g * GROUP
        return pltpu.make_async_copy(
            x_hbm.at[pl.ds(row0 * N, GN)],
            xbuf.at[pl.ds(buf_i * GN, GN)], sem)

    copy_in(0, 0, sem0).start()

    def group_body(g, carry):
        parity = g & 1

        @pl.when(parity == 0)
        def _():
            copy_in(g, 0, sem0).wait()

        @pl.when(parity == 1)
        def _():
            copy_in(g, 1, sem1).wait()

        @pl.when((g + 1 < NGRP) & (parity == 0))
        def _():
            copy_in(g + 1, 1, sem1).start()

        @pl.when((g + 1 < NGRP) & (parity == 1))
        def _():
            copy_in(g + 1, 0, sem0).start()

        boff = parity * GN

        def key_body(j, c):
            for u in range(4):
                off = j * (8 * L) + u * (2 * L)
                a = xbuf[pl.ds(boff + off, L)]
                b = xbuf[pl.ds(boff + off + L, L)]
                p = plsc.pack(a, b, format=plsc.PackFormat.INTERLEAVED)
                kbuf[pl.ds(off // 2, L)] = plsc.bitcast(p, jnp.int32)
            return c

        lax.fori_loop(0, GROUP * N // (8 * L), key_body, 0)

        def row_body(r, ovec):
            kbase = r * (N // 2)
            base = r * N

            t1v, t2v = zero, zero
            for i in range(NBITS):
                bitc = 32768 >> i
                cand1 = t1v + bitc
                cand2 = t2v + bitc
                cv1 = u2bf(cand1)
                cv2 = u2bf(65535 - cand2)

                def cbody(j, c, cv1=cv1, cv2=cv2):
                    c1a, c1b, c2a, c2b = c
                    for u in range(8):
                        v = plsc.bitcast(
                            kbuf[pl.ds(kbase + (j * 8 + u) * L, L)],
                            jnp.bfloat16)
                        i1 = jnp.where(v >= cv1, bone, bzero)
                        i2 = jnp.where(v <= cv2, bone, bzero)
                        if u % 2 == 0:
                            c1a = c1a + i1
                            c2a = c2a + i2
                        else:
                            c1b = c1b + i1
                            c2b = c2b + i2
                    return c1a, c1b, c2a, c2b

                c1a, c1b, c2a, c2b = lax.fori_loop(
                    0, CH32 // 8, cbody, (bzero, bzero, bzero, bzero))
                u1a, u1b = plsc.unpack(c1a + c1b,
                                       format=plsc.PackFormat.INTERLEAVED)
                u2a, u2b = plsc.unpack(c2a + c2b,
                                       format=plsc.PackFormat.INTERLEAVED)
                n1 = allsum(u1a + u1b)
                n2 = allsum(u2a + u2b)
                t1v = jnp.where(n1 >= float(K), cand1, t1v)
                t2v = jnp.where(n2 >= float(K), cand2, t2v)
            t1, t2 = t1v, t2v
            bot = 65535 - t2

            val_top = 0.5 * (u2f32(t1) + u2f32(t1 + 3))
            val_bot = 0.5 * (u2f32(bot - 3) + u2f32(bot))
            ub = 0.5 * (u2f32(t1 + 3) + u2f32(t1 + 4))
            lb = 0.5 * (u2f32(bot - 4) + u2f32(bot - 3))

            def fbody(j, c):
                cg, sg, cl, sl = c
                for u in range(8):
                    xv = xbuf[pl.ds(boff + base + (j * 8 + u) * L, L)]
                    m1 = xv > ub
                    m2 = xv < lb
                    cg = cg + jnp.where(m1, one, zero)
                    sg = sg + jnp.where(m1, xv, fzero)
                    cl = cl + jnp.where(m2, one, zero)
                    sl = sl + jnp.where(m2, xv, fzero)
                return cg, sg, cl, sl

            cg, sg, cl, sl = lax.fori_loop(
                0, N // (8 * L), fbody, (zero, fzero, zero, fzero))

            ng = float(K) - allsum(cg.astype(jnp.float32))
            nl = float(K) - allsum(cl.astype(jnp.float32))
            sgv = allsum(sg)
            slv = allsum(sl)
            top_sum = sgv + ng * val_top
            bot_sum = slv + nl * val_bot
            outv = top_sum * (1.0 / (2 * K)) + bot_sum * (ALPHA / (2 * K))
            return jnp.where(lanes == r, outv, ovec)

        ovec = lax.fori_loop(0, GROUP, row_body, fzero)
        outbuf[pl.ds(g * GROUP, GROUP)] = ovec
        return carry

    lax.fori_loop(0, NGRP, group_body, 0)
    pltpu.sync_copy(outbuf, out_hbm.at[pl.ds(wid * RPW, RPW)])


@jax.jit
def kernel(input):
    x = input.reshape(R * N)
    mesh = plsc.VectorSubcoreMesh(
        core_axis_name="c", subcore_axis_name="s",
        num_cores=NC, num_subcores=NS)
    out = pl.kernel(
        _kernel_body,
        out_type=jax.ShapeDtypeStruct((R,), jnp.float32),
        mesh=mesh,
        compiler_params=pltpu.CompilerParams(needs_layout_passes=False),
        scratch_types=[
            pltpu.VMEM((2 * GROUP * N,), jnp.float32),
            pltpu.VMEM((GROUP * N // 2,), jnp.int32),
            pltpu.VMEM((RPW,), jnp.float32),
            pltpu.SemaphoreType.DMA,
            pltpu.SemaphoreType.DMA,
        ],
    )(x)
    return out.reshape(B, C)
